# baseline (speedup 1.0000x reference)
import functools

import jax
import jax.numpy as jnp
from jax.experimental import pallas as pl
from jax.experimental.pallas import tpu as pltpu

_TOKENS = 32768
_D_MODEL = 4096
_NUM_EXPERTS = 64
_TOP_K = 8
_BT = 1024


_CH = 128


def _router_body(x_ref, w_ref, logits_ref, probs_ref, topw_ref, topi_ref):
    l = jnp.dot(x_ref[...], w_ref[...], preferred_element_type=jnp.float32)
    logits_ref[...] = l

    invcols = jax.lax.broadcasted_iota(jnp.int32, (_CH, _NUM_EXPERTS), 1)
    invcols = (_NUM_EXPERTS - 1) - invcols
    for c in range(_BT // _CH):
        rows = pl.ds(c * _CH, _CH)
        lc = logits_ref[rows, :]
        m = jnp.max(lc, axis=1, keepdims=True)
        ex = jnp.exp(lc - m)
        p = ex / jnp.sum(ex, axis=1, keepdims=True)
        probs_ref[rows, :] = p

        v = p
        ws = []
        ids = []
        for _ in range(_TOP_K):
            mj = jnp.max(v, axis=1, keepdims=True)
            t = jnp.where(v == mj, invcols, -1)
            am = jnp.max(t, axis=1, keepdims=True)
            ws.append(mj)
            ids.append((_NUM_EXPERTS - 1) - am)
            v = jnp.where(t == am, -1.0, v)
        topw_ref[rows, :] = jnp.concatenate(ws, axis=1)
        topi_ref[rows, :] = jnp.concatenate(ids, axis=1)


@jax.jit
def kernel(x, W):
    grid = (_TOKENS // _BT,)
    out_shapes = (
        jax.ShapeDtypeStruct((_TOKENS, _NUM_EXPERTS), jnp.float32),
        jax.ShapeDtypeStruct((_TOKENS, _NUM_EXPERTS), jnp.float32),
        jax.ShapeDtypeStruct((_TOKENS, _TOP_K), jnp.float32),
        jax.ShapeDtypeStruct((_TOKENS, _TOP_K), jnp.int32),
    )
    logits, probs, topw, topi = pl.pallas_call(
        _router_body,
        grid=grid,
        in_specs=[
            pl.BlockSpec((_BT, _D_MODEL), lambda i: (i, 0)),
            pl.BlockSpec((_D_MODEL, _NUM_EXPERTS), lambda i: (0, 0)),
        ],
        out_specs=(
            pl.BlockSpec((_BT, _NUM_EXPERTS), lambda i: (i, 0)),
            pl.BlockSpec((_BT, _NUM_EXPERTS), lambda i: (i, 0)),
            pl.BlockSpec((_BT, _TOP_K), lambda i: (i, 0)),
            pl.BlockSpec((_BT, _TOP_K), lambda i: (i, 0)),
        ),
        out_shape=out_shapes,
        compiler_params=pltpu.CompilerParams(
            dimension_semantics=("arbitrary",),
        ),
    )(x, W)
    return logits, probs, topw, topi


# speedup vs baseline: 1.2792x; 1.2792x over previous
import functools

import jax
import jax.numpy as jnp
from jax.experimental import pallas as pl
from jax.experimental.pallas import tpu as pltpu

_TOKENS = 32768
_D_MODEL = 4096
_NUM_EXPERTS = 64
_TOP_K = 8
_BT = 1024
_CH = 128


def _router_body(x_ref, w_ref, logits_ref, probs_ref, topw_ref, topi_ref):
    l = jnp.dot(x_ref[...], w_ref[...], preferred_element_type=jnp.float32)
    logits_ref[...] = l

    invrows = jax.lax.broadcasted_iota(jnp.int32, (_NUM_EXPERTS, _CH), 0)
    invrows = (_NUM_EXPERTS - 1) - invrows
    for c in range(_BT // _CH):
        rows = pl.ds(c * _CH, _CH)
        lt = logits_ref[rows, :].T

        m0 = jnp.max(lt, axis=0, keepdims=True)
        m0b = jnp.broadcast_to(m0, (_NUM_EXPERTS, _CH))
        ex = jnp.exp(lt - m0b)
        s = jnp.sum(ex, axis=0, keepdims=True)
        rs = 1.0 / s
        probs_ref[rows, :] = (ex * jnp.broadcast_to(rs, (_NUM_EXPERTS, _CH))).T

        v = lt
        ls = []
        ids = []
        for j in range(_TOP_K):
            mj = m0 if j == 0 else jnp.max(v, axis=0, keepdims=True)
            mjb = jnp.broadcast_to(mj, (_NUM_EXPERTS, _CH))
            t = jnp.where(v == mjb, invrows, -1)
            am = jnp.max(t, axis=0, keepdims=True)
            ls.append(mj)
            ids.append((_NUM_EXPERTS - 1) - am)
            v = jnp.where(t == jnp.broadcast_to(am, (_NUM_EXPERTS, _CH)), -jnp.inf, v)
        lsel = jnp.concatenate(ls, axis=0)
        cols = pl.ds(c * _CH, _CH)
        topw_ref[:, cols] = jnp.exp(lsel - jnp.broadcast_to(m0, (_TOP_K, _CH))) * (
            jnp.broadcast_to(rs, (_TOP_K, _CH))
        )
        topi_ref[:, cols] = jnp.concatenate(ids, axis=0)


@jax.jit
def kernel(x, W):
    grid = (_TOKENS // _BT,)
    out_shapes = (
        jax.ShapeDtypeStruct((_TOKENS, _NUM_EXPERTS), jnp.float32),
        jax.ShapeDtypeStruct((_TOKENS, _NUM_EXPERTS), jnp.float32),
        jax.ShapeDtypeStruct((_TOP_K, _TOKENS), jnp.float32),
        jax.ShapeDtypeStruct((_TOP_K, _TOKENS), jnp.int32),
    )
    logits, probs, topw_t, topi_t = pl.pallas_call(
        _router_body,
        grid=grid,
        in_specs=[
            pl.BlockSpec((_BT, _D_MODEL), lambda i: (i, 0)),
            pl.BlockSpec((_D_MODEL, _NUM_EXPERTS), lambda i: (0, 0)),
        ],
        out_specs=(
            pl.BlockSpec((_BT, _NUM_EXPERTS), lambda i: (i, 0)),
            pl.BlockSpec((_BT, _NUM_EXPERTS), lambda i: (i, 0)),
            pl.BlockSpec((_TOP_K, _BT), lambda i: (0, i)),
            pl.BlockSpec((_TOP_K, _BT), lambda i: (0, i)),
        ),
        out_shape=out_shapes,
        compiler_params=pltpu.CompilerParams(
            dimension_semantics=("arbitrary",),
        ),
    )(x, W)
    return logits, probs, topw_t.T, topi_t.T
